# baseline (speedup 1.0000x reference)
import functools

import jax
import jax.numpy as jnp
from jax import lax
from jax.experimental import pallas as pl
from jax.experimental.pallas import tpu as pltpu
from jax.experimental.pallas import tpu_sc as plsc

_VOCAB = 1_000_000
_EMB = 64
_B = 4096
_L = 50
_SCALE = 8.0

_NUM_CORES = 2
_NUM_SUBCORES = 16
_LANES = 16
_NW = _NUM_CORES * _NUM_SUBCORES

_B_PER_W = _B // _NW
_RB = 16
_NR = _B_PER_W // _RB
_LSPLIT = _L // 2
_GC = 80
_N_GC = _B_PER_W * _L // _GC
_GC_PER_HALF = _LSPLIT * _RB // _GC

_TBLK = 16384


def _make_tc_widen_kernel():
  def body(tt_ref, wide_ref):
    t = tt_ref[...].T
    wide_ref[...] = jnp.concatenate([t, t], axis=1)

  return pl.pallas_call(
      body,
      grid=(pl.cdiv(_VOCAB, _TBLK),),
      in_specs=[pl.BlockSpec((_EMB, _TBLK), lambda i: (0, i))],
      out_specs=pl.BlockSpec((_TBLK, 2 * _EMB), lambda i: (i, 0)),
      out_shape=jax.ShapeDtypeStruct((_VOCAB, 2 * _EMB), jnp.float32),
  )


_tc_widen = _make_tc_widen_kernel()


def _make_sc_kernel():
  mesh = plsc.VectorSubcoreMesh(
      core_axis_name="c",
      subcore_axis_name="s",
      num_cores=_NUM_CORES,
      num_subcores=_NUM_SUBCORES,
  )

  n_half = 2 * _NR
  rows_half = _LSPLIT * _RB

  @functools.partial(
      pl.kernel,
      out_type=jax.ShapeDtypeStruct((_L, _EMB, _B), jnp.float32),
      mesh=mesh,
      compiler_params=pltpu.CompilerParams(
          use_tc_tiling_on_sc=False, needs_layout_passes=False),
      scratch_types=[
          pltpu.VMEM((_N_GC, _GC), jnp.int32),
          pltpu.VMEM((2, rows_half, _EMB), jnp.float32),
          pltpu.VMEM((2, _LSPLIT, _EMB, _RB), jnp.float32),
          pltpu.SemaphoreType.DMA,
          pltpu.SemaphoreType.DMA,
          pltpu.SemaphoreType.DMA,
          pltpu.SemaphoreType.DMA,
      ],
  )
  def emb_kernel(idx_hbm, table_hbm, out_hbm, idx_v, gbuf, wbuf,
                 gsem0, gsem1, wsem0, wsem1):
    gsems = (gsem0, gsem1)
    wsems = (wsem0, wsem1)
    wid = lax.axis_index("s") * _NUM_CORES + lax.axis_index("c")
    b0w = wid * _B_PER_W

    pltpu.sync_copy(idx_hbm.at[wid], idx_v)
    def dbl_row(g, carry):
      for col in range(_GC // _LANES):
        sl = pl.ds(col * _LANES, _LANES)
        idx_v[g, sl] = idx_v[g, sl] * 2
      return carry

    lax.fori_loop(0, _N_GC, dbl_row, 0)

    iota16 = lax.iota(jnp.int32, _LANES)

    def gather_start(h, s):
      for k in range(_GC_PER_HALF):
        g = h * _GC_PER_HALF + k
        pltpu.async_copy(
            table_hbm.at[idx_v.at[g]],
            gbuf.at[s, pl.ds(k * _GC, _GC)],
            gsems[s],
        )

    def gather_wait(s):
      for k in range(_GC_PER_HALF):
        pltpu.make_async_copy(
            table_hbm.at[idx_v.at[0]],
            gbuf.at[s, pl.ds(0, _GC)],
            gsems[s],
        ).wait()

    def transpose_scale(s):
      def per_l(l, carry):
        rows = l * _RB + iota16
        for c in range(_EMB):
          vals = plsc.load_gather(gbuf.at[s], [rows, jnp.full((_LANES,), c, jnp.int32)])
          wbuf[s, l, c, :] = vals * _SCALE
        return carry

      lax.fori_loop(0, _LSPLIT, per_l, 0)

    def write_start(h, s):
      r = h // 2
      l0 = (h % 2) * _LSPLIT
      bg = b0w + r * _RB
      pltpu.async_copy(
          wbuf.at[s],
          out_hbm.at[pl.ds(l0, _LSPLIT), :, pl.ds(bg, _RB)],
          wsems[s],
      )

    def write_wait(s):
      pltpu.make_async_copy(
          wbuf.at[s],
          out_hbm.at[pl.ds(0, _LSPLIT), :, pl.ds(0, _RB)],
          wsems[s],
      ).wait()

    gather_start(0, 0)
    gather_start(1, 1)
    for h in range(2):
      s = h % 2
      gather_wait(s)
      transpose_scale(s)
      gather_start(h + 2, s)
      write_start(h, s)

    def steady(t, carry):
      h0 = 2 + t * 2
      for s in range(2):
        h = h0 + s
        gather_wait(s)
        write_wait(s)
        transpose_scale(s)
        gather_start(h + 2, s)
        write_start(h, s)
      return carry

    lax.fori_loop(0, (n_half - 4) // 2, steady, 0)

    for h in range(n_half - 2, n_half):
      s = h % 2
      gather_wait(s)
      write_wait(s)
      transpose_scale(s)
      write_start(h, s)
    for s in range(2):
      write_wait(s)

  return emb_kernel


_emb_kernel = _make_sc_kernel()


@jax.jit
def kernel(tokens, table):
  wide = _tc_widen(table.T)
  flat = wide.reshape(2 * _VOCAB, _EMB)
  idx = (
      tokens.reshape(_NW, _NR, _RB, _L)
      .transpose(0, 1, 3, 2)
      .reshape(_NW, _N_GC, _GC)
  )
  out_t = _emb_kernel(idx, flat)
  return out_t.transpose(2, 0, 1)


# speedup vs baseline: 1.5161x; 1.5161x over previous
import functools

import jax
import jax.numpy as jnp
from jax import lax
from jax.experimental import pallas as pl
from jax.experimental.pallas import tpu as pltpu
from jax.experimental.pallas import tpu_sc as plsc

_VOCAB = 1_000_000
_EMB = 64
_B = 4096
_L = 50
_SCALE = 8.0

_NUM_CORES = 2
_NUM_SUBCORES = 16
_LANES = 16
_NW = _NUM_CORES * _NUM_SUBCORES

_B_TOTAL = _B * _L
_B_PER_W = _B_TOTAL // _NW
_CHUNK = 128
_N_CHUNKS = _B_PER_W // _CHUNK
_NBUF = 2


_TBLK = 16384


def _make_tc_widen_kernel():
  def body(tt_ref, wide_ref):
    t = tt_ref[...].T
    wide_ref[...] = jnp.concatenate([t, t], axis=1)

  return pl.pallas_call(
      body,
      grid=(pl.cdiv(_VOCAB, _TBLK),),
      in_specs=[pl.BlockSpec((_EMB, _TBLK), lambda i: (0, i))],
      out_specs=pl.BlockSpec((_TBLK, 2 * _EMB), lambda i: (i, 0)),
      out_shape=jax.ShapeDtypeStruct((_VOCAB, 2 * _EMB), jnp.float32),
  )


_tc_widen = _make_tc_widen_kernel()


def _make_sc_kernel():
  mesh = plsc.VectorSubcoreMesh(
      core_axis_name="c",
      subcore_axis_name="s",
      num_cores=_NUM_CORES,
      num_subcores=_NUM_SUBCORES,
  )

  @functools.partial(
      pl.kernel,
      out_type=jax.ShapeDtypeStruct((_B_TOTAL, _EMB), jnp.float32),
      mesh=mesh,
      compiler_params=pltpu.CompilerParams(use_tc_tiling_on_sc=False),
      scratch_types=[
          pltpu.VMEM((_N_CHUNKS, _CHUNK), jnp.int32),
          pltpu.VMEM((_NBUF, _CHUNK, _EMB), jnp.float32),
          pltpu.VMEM((_NBUF, _CHUNK, _EMB), jnp.float32),
          pltpu.SemaphoreType.DMA,
          pltpu.SemaphoreType.DMA,
          pltpu.SemaphoreType.DMA,
          pltpu.SemaphoreType.DMA,
      ],
  )
  def emb_kernel(idx_hbm, table_hbm, out_hbm, idx_v, gbuf, wbuf,
                 gsem0, gsem1, wsem0, wsem1):
    gsems = (gsem0, gsem1)
    wsems = (wsem0, wsem1)
    wid = lax.axis_index("s") * _NUM_CORES + lax.axis_index("c")
    base = wid * _CHUNK

    pltpu.sync_copy(idx_hbm.at[wid], idx_v)
    def dbl_row(g, carry):
      for col in range(_CHUNK // _LANES):
        sl = pl.ds(col * _LANES, _LANES)
        idx_v[g, sl] = idx_v[g, sl] * 2
      return carry

    lax.fori_loop(0, _N_CHUNKS, dbl_row, 0)

    def gather_start(g, b):
      pltpu.async_copy(table_hbm.at[idx_v.at[g]], gbuf.at[b], gsems[b])

    def gather_wait(b):
      pltpu.make_async_copy(table_hbm.at[idx_v.at[0]], gbuf.at[b],
                            gsems[b]).wait()

    def scale(b):
      def row(r, carry):
        for col in range(_EMB // _LANES):
          sl = pl.ds(col * _LANES, _LANES)
          wbuf[b, r, sl] = gbuf[b, r, sl] * _SCALE
        return carry
      lax.fori_loop(0, _CHUNK, row, 0)

    def write_start(g, b):
      off = g * _B + base
      pltpu.async_copy(wbuf.at[b], out_hbm.at[pl.ds(off, _CHUNK)], wsems[b])

    def write_wait(b):
      pltpu.make_async_copy(wbuf.at[b], out_hbm.at[pl.ds(0, _CHUNK)],
                            wsems[b]).wait()

    for b in range(_NBUF):
      gather_start(b, b)
    for i in range(_NBUF):
      b = i % _NBUF
      gather_wait(b)
      scale(b)
      gather_start(i + _NBUF, b)
      write_start(i, b)

    n_steady = (_N_CHUNKS - 2 * _NBUF) // _NBUF

    def steady(t, carry):
      i0 = _NBUF + t * _NBUF
      for b in range(_NBUF):
        i = i0 + b
        gather_wait(b)
        write_wait(b)
        scale(b)
        gather_start(i + _NBUF, b)
        write_start(i, b)
      return carry

    lax.fori_loop(0, n_steady, steady, 0)

    for i in range(_N_CHUNKS - _NBUF, _N_CHUNKS):
      b = i % _NBUF
      gather_wait(b)
      write_wait(b)
      scale(b)
      write_start(i, b)
    for b in range(_NBUF):
      write_wait(b)

  return emb_kernel


_emb_kernel = _make_sc_kernel()


@jax.jit
def kernel(tokens, table):
  wide = _tc_widen(table.T)
  flat = wide.reshape(2 * _VOCAB, _EMB)
  idx = tokens.reshape(_NW, _CHUNK, _N_CHUNKS).transpose(0, 2, 1)
  out = _emb_kernel(idx, flat)
  return out.reshape(_L, _B, _EMB).transpose(1, 0, 2)
